# baseline (speedup 1.0000x reference)
import functools

import jax
import jax.numpy as jnp
from jax.experimental import pallas as pl
from jax.experimental.pallas import tpu as pltpu


def _cnn_block_kernel(x_ref, w_ref, b_ref, o_ref, xp_ref, col_ref,
                      *, H, W, Cin, Cout):
    Ho, Wo = H // 2, W // 2
    M = H * W

    xt = jnp.transpose(x_ref[...].astype(jnp.bfloat16), (1, 0))
    xi = xt.reshape(H, W, Cin)

    xp_ref[0:1, :, :] = jnp.zeros((1, W + 2, Cin), jnp.bfloat16)
    xp_ref[H + 1:H + 2, :, :] = jnp.zeros((1, W + 2, Cin), jnp.bfloat16)
    xp_ref[:, 0:1, :] = jnp.zeros((H + 2, 1, Cin), jnp.bfloat16)
    xp_ref[:, W + 1:W + 2, :] = jnp.zeros((H + 2, 1, Cin), jnp.bfloat16)
    xp_ref[1:H + 1, 1:W + 1, :] = xi
    xp = xp_ref[...]

    for dy in range(3):
        for dx in range(3):
            t = dy * 3 + dx
            col_ref[:, t * Cin:(t + 1) * Cin] = (
                xp[dy:dy + H, dx:dx + W, :].reshape(M, Cin))

    acc = jnp.dot(col_ref[...], w_ref[...], preferred_element_type=jnp.float32)

    acc = jnp.maximum(acc + b_ref[...], 0.0)

    wp = acc.reshape(H * Wo, 2, Cout)
    wp = jnp.maximum(wp[:, 0, :], wp[:, 1, :])
    hp = wp.reshape(Ho, 2, Wo, Cout)
    pooled = jnp.maximum(hp[:, 0], hp[:, 1])

    o_ref[...] = jnp.transpose(pooled.reshape(Ho * Wo, Cout), (1, 0))


def kernel(x_nchw, w_oihw, bias):
    B, Cin, H, W = x_nchw.shape
    Cout = w_oihw.shape[0]
    Ho, Wo = H // 2, W // 2
    K = 9 * Cin

    x_flat = x_nchw.reshape(B, Cin, H * W)
    w_mat = jnp.transpose(w_oihw, (2, 3, 1, 0)).reshape(K, Cout)
    w_mat = w_mat.astype(jnp.bfloat16)
    b_row = bias.reshape(1, Cout).astype(jnp.float32)

    body = functools.partial(_cnn_block_kernel, H=H, W=W, Cin=Cin, Cout=Cout)
    out_flat = pl.pallas_call(
        body,
        out_shape=jax.ShapeDtypeStruct((B, Cout, Ho * Wo), x_nchw.dtype),
        grid=(B,),
        in_specs=[
            pl.BlockSpec((None, Cin, H * W), lambda b: (b, 0, 0)),
            pl.BlockSpec((K, Cout), lambda b: (0, 0)),
            pl.BlockSpec((1, Cout), lambda b: (0, 0)),
        ],
        out_specs=pl.BlockSpec((None, Cout, Ho * Wo), lambda b: (b, 0, 0)),
        scratch_shapes=[
            pltpu.VMEM((H + 2, W + 2, Cin), jnp.bfloat16),
            pltpu.VMEM((H * W, K), jnp.bfloat16),
        ],
        compiler_params=pltpu.CompilerParams(
            dimension_semantics=("parallel",),
        ),
    )(x_flat, w_mat, b_row)

    return out_flat.reshape(B, Cout, Ho, Wo)


# speedup vs baseline: 1.1345x; 1.1345x over previous
import functools

import jax
import jax.numpy as jnp
from jax.experimental import pallas as pl
from jax.experimental.pallas import tpu as pltpu


def _cnn_block_kernel(x_ref, w_ref, b_ref, o_ref, xp_ref, col_ref,
                      *, H, W, Cin, Cout):
    Ho, Wo = H // 2, W // 2
    M = H * W
    X0 = 128

    xp_ref[:, 0:X0] = jnp.zeros((Cin, X0), jnp.float32)
    xp_ref[:, X0 + M:X0 + M + X0] = jnp.zeros((Cin, X0), jnp.float32)
    xp_ref[:, X0:X0 + M] = x_ref[...]
    xp = xp_ref[...]

    lane = jax.lax.broadcasted_iota(jnp.int32, (1, M), 1) % W
    not_first = lane != 0
    not_last = lane != (W - 1)

    zero_b = jnp.zeros((), jnp.bfloat16)
    for dy in range(3):
        for dx in range(3):
            t = dy * 3 + dx
            s = X0 + (dy - 1) * W + (dx - 1)
            v = xp[:, s:s + M].astype(jnp.bfloat16)
            if dx == 0:
                v = jnp.where(not_first, v, zero_b)
            elif dx == 2:
                v = jnp.where(not_last, v, zero_b)
            col_ref[t * Cin:(t + 1) * Cin, :] = v

    acc = jnp.dot(w_ref[...], col_ref[...], preferred_element_type=jnp.float32)

    at = jnp.transpose(acc, (1, 0))
    at = jnp.maximum(at + b_ref[...], 0.0)

    wp = at.reshape(H * Wo, 2, Cout)
    wp = jnp.maximum(wp[:, 0, :], wp[:, 1, :])
    h3 = wp.reshape(Ho, 2 * Wo, Cout)
    pooled = jnp.maximum(h3[:, 0:Wo, :], h3[:, Wo:2 * Wo, :])

    o_ref[...] = jnp.transpose(pooled.reshape(Ho * Wo, Cout), (1, 0))


def kernel(x_nchw, w_oihw, bias):
    B, Cin, H, W = x_nchw.shape
    Cout = w_oihw.shape[0]
    Ho, Wo = H // 2, W // 2
    K = 9 * Cin
    X0 = 128

    x_flat = x_nchw.reshape(B, Cin, H * W)
    w_mat = jnp.transpose(w_oihw, (0, 2, 3, 1)).reshape(Cout, K)
    w_mat = w_mat.astype(jnp.bfloat16)
    b_row = bias.reshape(1, Cout).astype(jnp.float32)

    body = functools.partial(_cnn_block_kernel, H=H, W=W, Cin=Cin, Cout=Cout)
    out_flat = pl.pallas_call(
        body,
        out_shape=jax.ShapeDtypeStruct((B, Cout, Ho * Wo), x_nchw.dtype),
        grid=(B,),
        in_specs=[
            pl.BlockSpec((None, Cin, H * W), lambda b: (b, 0, 0)),
            pl.BlockSpec((Cout, K), lambda b: (0, 0)),
            pl.BlockSpec((1, Cout), lambda b: (0, 0)),
        ],
        out_specs=pl.BlockSpec((None, Cout, Ho * Wo), lambda b: (b, 0, 0)),
        scratch_shapes=[
            pltpu.VMEM((Cin, X0 + H * W + X0), jnp.float32),
            pltpu.VMEM((K, H * W), jnp.bfloat16),
        ],
        compiler_params=pltpu.CompilerParams(
            dimension_semantics=("arbitrary",),
        ),
    )(x_flat, w_mat, b_row)

    return out_flat.reshape(B, Cout, Ho, Wo)
